# Initial kernel scaffold; baseline (speedup 1.0000x reference)
import jax
import jax.numpy as jnp
from jax.experimental import pallas as pl


def kernel(latent_z, gamma, sparse_i, sparse_j, sparse_w):
    raise NotImplementedError("write your pallas kernel here")


# speedup vs baseline: 140.7944x; 140.7944x over previous
import functools

import jax
import jax.numpy as jnp
from jax import lax
from jax.experimental import pallas as pl
from jax.experimental.pallas import tpu as pltpu
from jax.experimental.pallas import tpu_sc as plsc

N = 100000
E = 6400000
D = 8
ROW = 16
NC = 2
NS = 16
NW = NC * NS
EPW = E // NW
C = 1600
CPW = EPW // C
PAIRS = C // 16


def _sqrt16(x):
    xi = plsc.bitcast(x, jnp.int32)
    yi = jnp.int32(0x5F3759DF) - lax.shift_right_logical(xi, 1)
    y = plsc.bitcast(yi, jnp.float32)
    xh = x * jnp.float32(0.5)
    for _ in range(3):
        y = y * (jnp.float32(1.5) - xh * y * y)
    return x * y


_mesh = plsc.VectorSubcoreMesh(core_axis_name="c", subcore_axis_name="s")


@functools.partial(
    pl.kernel,
    mesh=_mesh,
    out_type=jax.ShapeDtypeStruct((NW, 16), jnp.float32),
    scratch_types=[
        pltpu.VMEM((C,), jnp.int32),
        pltpu.VMEM((C,), jnp.int32),
        pltpu.VMEM((C,), jnp.float32),
        pltpu.VMEM((C, ROW), jnp.float32),
        pltpu.VMEM((C, ROW), jnp.float32),
        pltpu.VMEM((16,), jnp.float32),
        pltpu.SemaphoreType.DMA,
        pltpu.SemaphoreType.DMA,
    ],
    compiler_params=pltpu.CompilerParams(
        needs_layout_passes=False, use_tc_tiling_on_sc=False),
)
def _lsm_sc(tab_hbm, si_hbm, sj_hbm, w_hbm, out_hbm,
            idxi_v, idxj_v, w_v, rowsi_v, rowsj_v, acc_v, semi, semj):
    wid = lax.axis_index("s") * NC + lax.axis_index("c")
    iota16 = lax.broadcasted_iota(jnp.int32, (16,), 0)

    def chunk_body(t, acc):
        base = wid * EPW + t * C
        pltpu.sync_copy(si_hbm.at[pl.ds(base, C)], idxi_v)
        pltpu.sync_copy(sj_hbm.at[pl.ds(base, C)], idxj_v)
        pltpu.sync_copy(w_hbm.at[pl.ds(base, C)], w_v)
        ci = pltpu.async_copy(tab_hbm.at[idxi_v], rowsi_v, semi)
        cj = pltpu.async_copy(tab_hbm.at[idxj_v], rowsj_v, semj)
        ci.wait()
        cj.wait()

        def pair_body(k, acc):
            e0 = k * 16
            ridx = e0 + iota16
            s = jnp.zeros((16,), jnp.float32)
            for d in range(D):
                cidx = jnp.full((16,), d, jnp.int32)
                a = plsc.load_gather(rowsi_v, [ridx, cidx])
                b = plsc.load_gather(rowsj_v, [ridx, cidx])
                diff = a - b + jnp.float32(1e-6)
                s = s + diff * diff
            c8 = jnp.full((16,), D, jnp.int32)
            gi = plsc.load_gather(rowsi_v, [ridx, c8])
            gj = plsc.load_gather(rowsj_v, [ridx, c8])
            r = gi + gj - _sqrt16(s) - w_v[pl.ds(e0, 16)]
            return acc + r * r

        return lax.fori_loop(0, PAIRS, pair_body, acc)

    acc = lax.fori_loop(0, CPW, chunk_body, jnp.zeros((16,), jnp.float32))
    acc_v[...] = acc
    pltpu.sync_copy(acc_v, out_hbm.at[wid])


def kernel(latent_z, gamma, sparse_i, sparse_j, sparse_w):
    tab = jnp.concatenate(
        [latent_z, gamma[:, None], jnp.zeros((N, ROW - D - 1), jnp.float32)],
        axis=1)
    partials = _lsm_sc(tab, sparse_i, sparse_j, sparse_w)
    return jnp.sum(partials)


# speedup vs baseline: 230.9017x; 1.6400x over previous
import functools

import jax
import jax.numpy as jnp
from jax import lax
from jax.experimental import pallas as pl
from jax.experimental.pallas import tpu as pltpu
from jax.experimental.pallas import tpu_sc as plsc

N = 100000
E = 6400000
D = 8
ROW = 16
NC = 2
NS = 16
NW = NC * NS
EPW = E // NW
C = 800
CPW = EPW // C
PAIRS = C // 16


def _sqrt16(x):
    xi = plsc.bitcast(x, jnp.int32)
    yi = jnp.int32(0x5F3759DF) - lax.shift_right_logical(xi, 1)
    y = plsc.bitcast(yi, jnp.float32)
    xh = x * jnp.float32(0.5)
    for _ in range(2):
        y = y * (jnp.float32(1.5) - xh * y * y)
    return x * y


_mesh = plsc.VectorSubcoreMesh(core_axis_name="c", subcore_axis_name="s")


@functools.partial(
    pl.kernel,
    mesh=_mesh,
    out_type=jax.ShapeDtypeStruct((NW, 16), jnp.float32),
    scratch_types=[
        [pltpu.VMEM((C,), jnp.int32)] * 2,
        [pltpu.VMEM((C,), jnp.int32)] * 2,
        [pltpu.VMEM((C,), jnp.float32)] * 2,
        [pltpu.VMEM((C, ROW), jnp.float32)] * 2,
        [pltpu.VMEM((C, ROW), jnp.float32)] * 2,
        pltpu.VMEM((16,), jnp.float32),
        [pltpu.SemaphoreType.DMA] * 2,
        [pltpu.SemaphoreType.DMA] * 2,
        [pltpu.SemaphoreType.DMA] * 2,
    ],
    compiler_params=pltpu.CompilerParams(
        needs_layout_passes=False, use_tc_tiling_on_sc=False),
)
def _lsm_sc(tab_hbm, si_hbm, sj_hbm, w_hbm, out_hbm,
            idxi_v, idxj_v, w_v, rowsi_v, rowsj_v, acc_v,
            semlin, semgat, semw):
    wid = lax.axis_index("s") * NC + lax.axis_index("c")
    iota16 = lax.broadcasted_iota(jnp.int32, (16,), 0)
    base0 = wid * EPW

    def lin_start(g, b):
        base = base0 + g * C
        pltpu.async_copy(si_hbm.at[pl.ds(base, C)], idxi_v[b], semlin[b])
        pltpu.async_copy(sj_hbm.at[pl.ds(base, C)], idxj_v[b], semlin[b])

    def lin_wait(b):
        pltpu.make_async_copy(si_hbm.at[pl.ds(0, C)], idxi_v[b], semlin[b]).wait()
        pltpu.make_async_copy(sj_hbm.at[pl.ds(0, C)], idxj_v[b], semlin[b]).wait()

    def gw_start(g, b):
        base = base0 + g * C
        pltpu.async_copy(tab_hbm.at[idxi_v[b]], rowsi_v[b], semgat[b])
        pltpu.async_copy(tab_hbm.at[idxj_v[b]], rowsj_v[b], semgat[b])
        pltpu.async_copy(w_hbm.at[pl.ds(base, C)], w_v[b], semw[b])

    def gw_wait(b):
        pltpu.make_async_copy(tab_hbm.at[idxi_v[b]], rowsi_v[b], semgat[b]).wait()
        pltpu.make_async_copy(tab_hbm.at[idxj_v[b]], rowsj_v[b], semgat[b]).wait()
        pltpu.make_async_copy(w_hbm.at[pl.ds(0, C)], w_v[b], semw[b]).wait()

    def compute(b, acc):
        ri, rj, wv = rowsi_v[b], rowsj_v[b], w_v[b]

        def pair_body(k, acc):
            e0 = k * 16
            ridx = e0 + iota16
            s = jnp.zeros((16,), jnp.float32)
            for d in range(D):
                cidx = jnp.full((16,), d, jnp.int32)
                a = plsc.load_gather(ri, [ridx, cidx])
                b_ = plsc.load_gather(rj, [ridx, cidx])
                diff = a - b_ + jnp.float32(1e-6)
                s = s + diff * diff
            c8 = jnp.full((16,), D, jnp.int32)
            gi = plsc.load_gather(ri, [ridx, c8])
            gj = plsc.load_gather(rj, [ridx, c8])
            r = gi + gj - _sqrt16(s) - wv[pl.ds(e0, 16)]
            return acc + r * r

        return lax.fori_loop(0, PAIRS, pair_body, acc)

    lin_start(0, 0)
    lin_wait(0)
    gw_start(0, 0)
    lin_start(1, 1)

    def step(t, acc):
        for b in (0, 1):
            g = t * 2 + b
            gw_wait(b)

            @pl.when(g + 1 < CPW)
            def _():
                lin_wait(1 - b)
                gw_start(g + 1, 1 - b)

            @pl.when(g + 2 < CPW)
            def _():
                lin_start(g + 2, b)

            acc = compute(b, acc)
        return acc

    acc = lax.fori_loop(0, CPW // 2, step, jnp.zeros((16,), jnp.float32))
    acc_v[...] = acc
    pltpu.sync_copy(acc_v, out_hbm.at[wid])


def kernel(latent_z, gamma, sparse_i, sparse_j, sparse_w):
    tab = jnp.concatenate(
        [latent_z, gamma[:, None], jnp.zeros((N, ROW - D - 1), jnp.float32)],
        axis=1)
    partials = _lsm_sc(tab, sparse_i, sparse_j, sparse_w)
    return jnp.sum(partials)
